# baseline (speedup 1.0000x reference)
import functools

import jax
import jax.numpy as jnp
from jax import lax
from jax.experimental import pallas as pl
from jax.experimental.pallas import tpu as pltpu
from jax.experimental.pallas import tpu_sc as plsc

_NC = 2
_NS = 16


def _sc_body(lhs_ref, starts_ref, nums_ref, out_ref, zbuf,
             bufs1_0, bufs1_1, bufs2_0, bufs2_1,
             vvec, idx1_0, idx1_1, idx2_0, idx2_1,
             zsem, gsem0, gsem1, wsem0, wsem1):
    wid = lax.axis_index("s") * _NC + lax.axis_index("c")
    b = wid // 4
    q = wid % 4

    iota = lax.iota(jnp.int32, 16)
    zeros16 = jnp.zeros((16,), jnp.float32)

    bufs1 = (bufs1_0, bufs1_1)
    bufs2 = (bufs2_0, bufs2_1)
    idx1 = (idx1_0, idx1_1)
    idx2 = (idx2_0, idx2_1)
    gsem = (gsem0, gsem1)
    wsem = (wsem0, wsem1)

    cs = pltpu.make_async_copy(starts_ref, vvec.at[pl.ds(0, 8)], gsem0)
    cn = pltpu.make_async_copy(nums_ref, vvec.at[pl.ds(8, 8)], gsem0)
    cs.start()
    cn.start()

    def _zrow(r, carry):
        for k in range(128):
            zbuf[r, pl.ds(k * 16, 16)] = zeros16
        return carry
    lax.fori_loop(0, 16, _zrow, 0)

    cs.wait()
    cn.wait()
    sv = vvec[...]
    narr = [sv[k + 8] for k in range(8)]
    s = jnp.int32(0)
    n = jnp.int32(0)
    for k in range(8):
        s = jnp.where(b == k, sv[k], s)
        n = jnp.where(b == k, narr[k], n)

    weights = []
    total = jnp.int32(384)
    for v in range(32):
        bv, qv = v // 4, v % 4
        av = jnp.int32(0)
        for cc in range(4):
            av = av + (narr[bv] > qv * 64 + 16 * cc).astype(jnp.int32)
        weights.append(4 + av)
        total = total + 4 + av
    share = (total + 31) // 32
    c_lo = jnp.int32(0)
    quota = jnp.int32(0)
    for v in range(32):
        qv_quota = jnp.maximum(share - weights[v], 0)
        c_lo = c_lo + jnp.where(v < wid, qv_quota, 0)
        quota = jnp.where(v == wid, qv_quota, quota)
    z_lo = jnp.minimum(c_lo, 384)
    z_hi = jnp.minimum(c_lo + quota, 384)

    def _zero_dma(g):
        b2 = g // 48
        r2 = pl.multiple_of(256 + (g % 48) * 16, 16)
        return pltpu.make_async_copy(
            zbuf, out_ref.at[b2, pl.ds(r2, 16), :], zsem)

    def _fire_zero(g, carry):
        _zero_dma(g).start()
        return carry
    lax.fori_loop(z_lo, z_hi, _fire_zero, 0)

    def n_of(c):
        return jnp.clip(n - (q * 64 + c * 16), 0, 16)

    def fire_gather(c):
        p = c % 2
        @pl.when(n_of(c) > 0)
        def _():
            base1 = b * 2048 + s + q * 64 + c * 16
            idx1[p][...] = base1 + iota
            idx2[p][...] = base1 + n + iota
            pltpu.make_async_copy(lhs_ref.at[idx1[p]], bufs1[p],
                                  gsem[p]).start()
            pltpu.make_async_copy(lhs_ref.at[idx2[p]], bufs2[p],
                                  gsem[p]).start()

    def write_copies(c):
        p = c % 2
        cb = q * 64 + c * 16
        w1 = pltpu.make_async_copy(
            bufs1[p], out_ref.at[b, pl.ds(cb, 16), pl.ds(0, 1024)], wsem[p])
        w2 = pltpu.make_async_copy(
            bufs2[p], out_ref.at[b, pl.ds(cb, 16), pl.ds(1024, 1024)],
            wsem[p])
        return w1, w2

    def drain_writes(c):
        @pl.when(n_of(c) > 0)
        def _():
            w1, w2 = write_copies(c)
            w1.wait()
            w2.wait()

    fire_gather(0)
    for c in range(4):
        if c + 1 < 4:
            if c - 1 >= 0:
                drain_writes(c - 1)
            fire_gather(c + 1)

        n_c = n_of(c)
        cb = q * 64 + c * 16
        p = c % 2

        @pl.when(n_c == 0)
        def _zero_chunk():
            pltpu.make_async_copy(
                zbuf, out_ref.at[b, pl.ds(cb, 16), :], zsem).start()

        @pl.when(n_c > 0)
        def _data_chunk():
            g1 = pltpu.make_async_copy(lhs_ref.at[idx1[p]], bufs1[p],
                                       gsem[p])
            g2 = pltpu.make_async_copy(lhs_ref.at[idx2[p]], bufs2[p],
                                       gsem[p])
            g1.wait()
            g2.wait()

            def _blank(r, carry):
                for k in range(64):
                    bufs1[p][r, pl.ds(k * 16, 16)] = zeros16
                    bufs2[p][r, pl.ds(k * 16, 16)] = zeros16
                return carry
            lax.fori_loop(n_c, 16, _blank, 0)

            w1, w2 = write_copies(c)
            w1.start()
            w2.start()

    drain_writes(2)
    drain_writes(3)

    def _drain_zero(g, carry):
        _zero_dma(g).wait()
        return carry
    lax.fori_loop(z_lo, z_hi, _drain_zero, 0)
    for c in range(4):
        cb = q * 64 + c * 16

        @pl.when(n_of(c) == 0)
        def _drain_zero_chunk():
            pltpu.make_async_copy(
                zbuf, out_ref.at[b, pl.ds(cb, 16), :], zsem).wait()


def kernel(last_hidden_state, start_marker_indices, num_marker_pairs):
    starts = start_marker_indices.astype(jnp.int32)
    nums = num_marker_pairs.astype(jnp.int32)

    mesh = plsc.VectorSubcoreMesh(core_axis_name="c", subcore_axis_name="s")
    sc = functools.partial(
        pl.kernel,
        mesh=mesh,
        out_type=jax.ShapeDtypeStruct((8, 1024, 2048), jnp.float32),
        scratch_types=[
            pltpu.VMEM((16, 2048), jnp.float32),
            pltpu.VMEM((16, 1024), jnp.float32),
            pltpu.VMEM((16, 1024), jnp.float32),
            pltpu.VMEM((16, 1024), jnp.float32),
            pltpu.VMEM((16, 1024), jnp.float32),
            pltpu.VMEM((16,), jnp.int32),
            pltpu.VMEM((16,), jnp.int32),
            pltpu.VMEM((16,), jnp.int32),
            pltpu.VMEM((16,), jnp.int32),
            pltpu.VMEM((16,), jnp.int32),
            pltpu.SemaphoreType.DMA,
            pltpu.SemaphoreType.DMA,
            pltpu.SemaphoreType.DMA,
            pltpu.SemaphoreType.DMA,
            pltpu.SemaphoreType.DMA,
        ],
    )(_sc_body)
    lhs_rows = last_hidden_state.reshape(16384, 1024)
    return sc(lhs_rows, starts, nums)


# speedup vs baseline: 1.0644x; 1.0644x over previous
import functools

import jax
import jax.numpy as jnp
from jax import lax
from jax.experimental import pallas as pl
from jax.experimental.pallas import tpu as pltpu
from jax.experimental.pallas import tpu_sc as plsc

_NC = 2
_NS = 16


def _sc_body(lhs_ref, starts_ref, nums_ref, out_ref, zbuf,
             bufs1_0, bufs1_1, bufs2_0, bufs2_1,
             vvec, idx1_0, idx1_1, idx2_0, idx2_1,
             zsem, gsem0, gsem1, wsem0, wsem1):
    wid = lax.axis_index("s") * _NC + lax.axis_index("c")
    b = wid // 4
    q = wid % 4

    iota = lax.iota(jnp.int32, 16)
    zeros16 = jnp.zeros((16,), jnp.float32)

    bufs1 = (bufs1_0, bufs1_1)
    bufs2 = (bufs2_0, bufs2_1)
    idx1 = (idx1_0, idx1_1)
    idx2 = (idx2_0, idx2_1)
    gsem = (gsem0, gsem1)
    wsem = (wsem0, wsem1)

    cs = pltpu.make_async_copy(starts_ref, vvec.at[pl.ds(0, 8)], gsem0)
    cn = pltpu.make_async_copy(nums_ref, vvec.at[pl.ds(8, 8)], gsem0)
    cs.start()
    cn.start()

    def _zrow(r, carry):
        for k in range(128):
            zbuf[r, pl.ds(k * 16, 16)] = zeros16
        return carry
    lax.fori_loop(0, 8, _zrow, 0)

    cs.wait()
    cn.wait()
    sv = vvec[...]
    s = jnp.int32(0)
    n = jnp.int32(0)
    for k in range(8):
        s = jnp.where(b == k, sv[k], s)
        n = jnp.where(b == k, sv[k + 8], n)

    zr0 = 256 + q * 192
    for k in range(24):
        pltpu.make_async_copy(
            zbuf, out_ref.at[b, pl.ds(zr0 + k * 8, 8), :], zsem).start()

    def n_of(c):
        return jnp.clip(n - (q * 64 + c * 16), 0, 16)

    def fire_gather(c):
        p = c % 2
        @pl.when(n_of(c) > 0)
        def _():
            base1 = b * 2048 + s + q * 64 + c * 16
            idx1[p][...] = base1 + iota
            idx2[p][...] = base1 + n + iota
            pltpu.make_async_copy(lhs_ref.at[idx1[p]], bufs1[p],
                                  gsem[p]).start()
            pltpu.make_async_copy(lhs_ref.at[idx2[p]], bufs2[p],
                                  gsem[p]).start()

    def write_copies(c):
        p = c % 2
        cb = q * 64 + c * 16
        w1 = pltpu.make_async_copy(
            bufs1[p], out_ref.at[b, pl.ds(cb, 16), pl.ds(0, 1024)], wsem[p])
        w2 = pltpu.make_async_copy(
            bufs2[p], out_ref.at[b, pl.ds(cb, 16), pl.ds(1024, 1024)],
            wsem[p])
        return w1, w2

    def drain_writes(c):
        @pl.when(n_of(c) > 0)
        def _():
            w1, w2 = write_copies(c)
            w1.wait()
            w2.wait()

    fire_gather(0)
    for c in range(4):
        if c + 1 < 4:
            if c - 1 >= 0:
                drain_writes(c - 1)
            fire_gather(c + 1)

        n_c = n_of(c)
        cb = q * 64 + c * 16
        p = c % 2

        @pl.when(n_c == 0)
        def _zero_chunk():
            pltpu.make_async_copy(
                zbuf, out_ref.at[b, pl.ds(cb, 8), :], zsem).start()
            pltpu.make_async_copy(
                zbuf, out_ref.at[b, pl.ds(cb + 8, 8), :], zsem).start()

        @pl.when(n_c > 0)
        def _data_chunk():
            g1 = pltpu.make_async_copy(lhs_ref.at[idx1[p]], bufs1[p],
                                       gsem[p])
            g2 = pltpu.make_async_copy(lhs_ref.at[idx2[p]], bufs2[p],
                                       gsem[p])
            g1.wait()
            g2.wait()

            def _blank(r, carry):
                for k in range(64):
                    bufs1[p][r, pl.ds(k * 16, 16)] = zeros16
                    bufs2[p][r, pl.ds(k * 16, 16)] = zeros16
                return carry
            lax.fori_loop(n_c, 16, _blank, 0)

            w1, w2 = write_copies(c)
            w1.start()
            w2.start()

    drain_writes(2)
    drain_writes(3)
    for k in range(24):
        pltpu.make_async_copy(
            zbuf, out_ref.at[b, pl.ds(zr0 + k * 8, 8), :], zsem).wait()
    for c in range(4):
        cb = q * 64 + c * 16

        @pl.when(n_of(c) == 0)
        def _drain_zero_chunk():
            pltpu.make_async_copy(
                zbuf, out_ref.at[b, pl.ds(cb, 8), :], zsem).wait()
            pltpu.make_async_copy(
                zbuf, out_ref.at[b, pl.ds(cb + 8, 8), :], zsem).wait()


def kernel(last_hidden_state, start_marker_indices, num_marker_pairs):
    starts = start_marker_indices.astype(jnp.int32)
    nums = num_marker_pairs.astype(jnp.int32)

    mesh = plsc.VectorSubcoreMesh(core_axis_name="c", subcore_axis_name="s")
    sc = functools.partial(
        pl.kernel,
        mesh=mesh,
        out_type=jax.ShapeDtypeStruct((8, 1024, 2048), jnp.float32),
        scratch_types=[
            pltpu.VMEM((8, 2048), jnp.float32),
            pltpu.VMEM((16, 1024), jnp.float32),
            pltpu.VMEM((16, 1024), jnp.float32),
            pltpu.VMEM((16, 1024), jnp.float32),
            pltpu.VMEM((16, 1024), jnp.float32),
            pltpu.VMEM((16,), jnp.int32),
            pltpu.VMEM((16,), jnp.int32),
            pltpu.VMEM((16,), jnp.int32),
            pltpu.VMEM((16,), jnp.int32),
            pltpu.VMEM((16,), jnp.int32),
            pltpu.SemaphoreType.DMA,
            pltpu.SemaphoreType.DMA,
            pltpu.SemaphoreType.DMA,
            pltpu.SemaphoreType.DMA,
            pltpu.SemaphoreType.DMA,
        ],
    )(_sc_body)
    lhs_rows = last_hidden_state.reshape(16384, 1024)
    return sc(lhs_rows, starts, nums)
